# Initial kernel scaffold; baseline (speedup 1.0000x reference)
import jax
import jax.numpy as jnp
from jax.experimental import pallas as pl


def kernel(x, edge_index, edge_attr, vert_emb, W_edge_emb, W_el, W_rel, b_rel, W_root, W_res, W_h1, b_h1, W_h2, b_h2):
    raise NotImplementedError("write your pallas kernel here")


# speedup vs baseline: 2.7104x; 2.7104x over previous
import functools
import math

import jax
import jax.numpy as jnp
from jax import lax
from jax.experimental import pallas as pl
from jax.experimental.pallas import tpu as pltpu
from jax.experimental.pallas import tpu_sc as plsc

N = 10000
E = 320000
H = 128
L = 5
NFF = 512

NC = 2
NS = 16
NW = NC * NS
EPW = E // NW
CH = 80
NCHUNK = EPW // CH
NPAD = 10240
RPT = NPAD // NS
ZR = 64
RPW = NPAD // NW


def _edge_pass(h_hbm, src_hbm, dst_hbm, ea_hbm, wct_hbm, out_hbm,
               srcidx, dstidx, attr, rows, msg, wct_v, zbuf, agg, gsem, ssem):
    cid = lax.axis_index("c")
    sid = lax.axis_index("s")
    wid = sid * NC + cid

    def zrow(r, carry):
        for v in range(8):
            zbuf[r, pl.ds(v * 16, 16)] = jnp.zeros((16,), jnp.float32)
        return carry
    lax.fori_loop(0, ZR, zrow, 0)

    def zblk(b, carry):
        pltpu.sync_copy(zbuf, agg.at[pl.ds(sid * RPT + b * ZR, ZR), :])
        return carry
    lax.fori_loop(0, RPT // ZR, zblk, 0)

    pltpu.sync_copy(wct_hbm, wct_v)
    plsc.subcore_barrier()
    wcs = tuple(tuple(wct_v[k, pl.ds(v * 16, 16)] for v in range(8))
                for k in range(4))

    def chunk_body(j, wcs):
        base = wid * EPW + j * CH
        pltpu.sync_copy(src_hbm.at[pl.ds(base, CH)], srcidx)
        pltpu.sync_copy(dst_hbm.at[pl.ds(base, CH)], dstidx)
        pltpu.sync_copy(ea_hbm.at[pl.ds(base * 4, CH * 4)],
                        attr.at[pl.ds(0, CH * 4)])
        pltpu.async_copy(h_hbm.at[srcidx], rows, gsem).wait()

        def ebody(c, wcs):
            av = attr[pl.ds(c * 4, 16)]
            a0 = av[0]
            a1 = av[1]
            a2 = av[2]
            a3 = av[3]
            for v in range(8):
                sl = pl.ds(v * 16, 16)
                ea = (a0 * wcs[0][v] + a1 * wcs[1][v]
                      + a2 * wcs[2][v] + a3 * wcs[3][v])
                msg[c, sl] = rows[c, sl] * ea
            return wcs
        wcs = lax.fori_loop(0, CH, ebody, wcs)

        pltpu.async_copy(msg, agg.at[dstidx], ssem, add=True).wait()
        return wcs

    lax.fori_loop(0, NCHUNK, chunk_body, wcs)
    plsc.subcore_barrier()
    pltpu.sync_copy(agg.at[pl.ds(sid * RPT, RPT), :],
                    out_hbm.at[cid, pl.ds(sid * RPT, RPT), :])


def _embed(tab_hbm, idx_hbm, out_hbm, idxv, rowsv, sem):
    cid = lax.axis_index("c")
    sid = lax.axis_index("s")
    wid = sid * NC + cid
    for j in range(RPW // CH):
        base = wid * RPW + j * CH
        pltpu.sync_copy(idx_hbm.at[pl.ds(base, CH)], idxv)
        pltpu.async_copy(tab_hbm.at[idxv], rowsv, sem).wait()
        pltpu.sync_copy(rowsv, out_hbm.at[pl.ds(base, CH), :])


def _fold(wemb_ref, wel_ref, out_ref):
    for l in range(L):
        out_ref[l] = lax.dot_general(
            wemb_ref[...], wel_ref[l], (((0,), (1,)), ((), ())),
            preferred_element_type=jnp.float32)


def _layer(p_ref, h_ref, wrel_ref, brel_ref, wroot_ref, wres_ref, out_ref):
    agg = p_ref[0] + p_ref[1]
    h = h_ref[...]
    t = lax.dot_general(agg, wrel_ref[...], (((1,), (1,)), ((), ())),
                        preferred_element_type=jnp.float32) + brel_ref[...]
    t = t + lax.dot_general(h, wroot_ref[...], (((1,), (1,)), ((), ())),
                            preferred_element_type=jnp.float32)
    t = jnp.maximum(t, 0.0)
    out_ref[...] = t + lax.dot_general(h, wres_ref[...], (((1,), (1,)), ((), ())),
                                       preferred_element_type=jnp.float32)


def _head(h_ref, w1_ref, b1_ref, w2_ref, b2_ref, out_ref):
    t = lax.dot_general(h_ref[...], w1_ref[...], (((1,), (1,)), ((), ())),
                        preferred_element_type=jnp.float32) + b1_ref[...]
    g = 0.5 * t * (1.0 + lax.erf(t * (1.0 / math.sqrt(2.0))))
    out_ref[...] = lax.dot_general(g, w2_ref[...], (((1,), (1,)), ((), ())),
                                   preferred_element_type=jnp.float32) + b2_ref[...]


BM = 1000


def kernel(x, edge_index, edge_attr, vert_emb, W_edge_emb, W_el, W_rel, b_rel,
           W_root, W_res, W_h1, b_h1, W_h2, b_h2):
    f32 = jnp.float32
    x_pad = jnp.pad(x.astype(jnp.int32), (0, NPAD - N))
    src = edge_index[0].astype(jnp.int32)
    dst = edge_index[1].astype(jnp.int32)
    ea_flat = edge_attr.reshape(E * 4)

    wct_all = pl.pallas_call(
        _fold,
        out_shape=jax.ShapeDtypeStruct((L, 4, H), f32),
    )(W_edge_emb, W_el)

    mesh = plsc.VectorSubcoreMesh(core_axis_name="c", subcore_axis_name="s",
                                  num_cores=NC, num_subcores=NS)

    embed_call = pl.kernel(
        _embed,
        out_type=jax.ShapeDtypeStruct((NPAD, H), f32),
        mesh=mesh,
        scratch_types=[
            pltpu.VMEM((CH,), jnp.int32),
            pltpu.VMEM((CH, H), f32),
            pltpu.SemaphoreType.DMA,
        ],
    )
    h = embed_call(vert_emb, x_pad)[:N]

    edge_call = pl.kernel(
        _edge_pass,
        out_type=jax.ShapeDtypeStruct((NC, NPAD, H), f32),
        mesh=mesh,
        scratch_types=[
            pltpu.VMEM((CH,), jnp.int32),
            pltpu.VMEM((CH,), jnp.int32),
            pltpu.VMEM((CH * 4 + 16,), f32),
            pltpu.VMEM((CH, H), f32),
            pltpu.VMEM((CH, H), f32),
            pltpu.VMEM((4, H), f32),
            pltpu.VMEM((ZR, H), f32),
            pltpu.VMEM_SHARED((NPAD, H), f32),
            pltpu.SemaphoreType.DMA,
            pltpu.SemaphoreType.DMA,
        ],
    )

    layer_call = pl.pallas_call(
        _layer,
        grid=(N // BM,),
        in_specs=[
            pl.BlockSpec((NC, BM, H), lambda m: (0, m, 0)),
            pl.BlockSpec((BM, H), lambda m: (m, 0)),
            pl.BlockSpec((H, H), lambda m: (0, 0)),
            pl.BlockSpec((1, H), lambda m: (0, 0)),
            pl.BlockSpec((H, H), lambda m: (0, 0)),
            pl.BlockSpec((H, H), lambda m: (0, 0)),
        ],
        out_specs=pl.BlockSpec((BM, H), lambda m: (m, 0)),
        out_shape=jax.ShapeDtypeStruct((N, H), f32),
    )

    for l in range(L):
        p = edge_call(h, src, dst, ea_flat, wct_all[l])
        h = layer_call(p, h, W_rel[l], b_rel[l].reshape(1, H),
                       W_root[l], W_res[l])

    w2_pad = jnp.pad(W_h2, ((0, H - 1), (0, 0)))
    b2_pad = jnp.pad(b2_col := b_h2.reshape(1, 1), ((0, 0), (0, H - 1)))
    y = pl.pallas_call(
        _head,
        grid=(N // BM,),
        in_specs=[
            pl.BlockSpec((BM, H), lambda m: (m, 0)),
            pl.BlockSpec((NFF, H), lambda m: (0, 0)),
            pl.BlockSpec((1, NFF), lambda m: (0, 0)),
            pl.BlockSpec((H, NFF), lambda m: (0, 0)),
            pl.BlockSpec((1, H), lambda m: (0, 0)),
        ],
        out_specs=pl.BlockSpec((BM, H), lambda m: (m, 0)),
        out_shape=jax.ShapeDtypeStruct((N, H), f32),
    )(h, W_h1, b_h1.reshape(1, NFF), w2_pad, b2_pad)
    return y[:, :1]


# speedup vs baseline: 2.9370x; 1.0836x over previous
import functools
import math

import jax
import jax.numpy as jnp
from jax import lax
from jax.experimental import pallas as pl
from jax.experimental.pallas import tpu as pltpu
from jax.experimental.pallas import tpu_sc as plsc

N = 10000
E = 320000
H = 128
L = 5
NFF = 512

NC = 2
NS = 16
NW = NC * NS
CH = 80
K = 2
KCH = K * CH
NSUPER = 64
EPW = KCH * NSUPER
E_PAD = NW * EPW
NPAD = 10240
RPT = NPAD // NS
ZR = 16
RPW = NPAD // NW


def _edge_pass(h_hbm, src_hbm, dst_hbm, ea_hbm, wct_hbm, out_hbm,
               srcidx0, srcidx1, dst2d0, dst2d1, attr0, attr1, rows0, rows1,
               wct_v, zbuf, agg,
               isem0, isem1, dsem0, dsem1, gsem0, gsem1, ssem0, ssem1):
    cid = lax.axis_index("c")
    sid = lax.axis_index("s")
    wid = sid * NC + cid
    srcidx = (srcidx0, srcidx1)
    dst2d = (dst2d0, dst2d1)
    attr = (attr0, attr1)
    rows = (rows0, rows1)
    isem = (isem0, isem1)
    dsem = (dsem0, dsem1)
    gsem = (gsem0, gsem1)
    ssem = (ssem0, ssem1)

    def zrow(r, carry):
        for v in range(8):
            zbuf[r, pl.ds(v * 16, 16)] = jnp.zeros((16,), jnp.float32)
        return carry
    lax.fori_loop(0, ZR, zrow, 0)

    def zblk(b, carry):
        pltpu.sync_copy(zbuf, agg.at[pl.ds(sid * RPT + b * ZR, ZR), :])
        return carry
    lax.fori_loop(0, RPT // ZR, zblk, 0)

    pltpu.sync_copy(wct_hbm, wct_v)
    plsc.subcore_barrier()

    ebase = wid * EPW

    def fire_src_attr(j, b):
        base = ebase + j * KCH
        pltpu.async_copy(src_hbm.at[pl.ds(base, KCH)], srcidx[b], isem[b])
        pltpu.async_copy(ea_hbm.at[pl.ds(base * 4, KCH * 4)],
                         attr[b].at[pl.ds(0, KCH * 4)], isem[b])

    def wait_src_attr(b):
        pltpu.make_async_copy(src_hbm.at[pl.ds(0, KCH)], srcidx[b],
                              isem[b]).wait()
        pltpu.make_async_copy(ea_hbm.at[pl.ds(0, KCH * 4)],
                              attr[b].at[pl.ds(0, KCH * 4)], isem[b]).wait()

    def fire_dst(j, b):
        base = ebase + j * KCH
        for k in range(K):
            pltpu.async_copy(dst_hbm.at[pl.ds(base + k * CH, CH)],
                             dst2d[b].at[k], dsem[b])

    def wait_dst(b):
        for k in range(K):
            pltpu.make_async_copy(dst_hbm.at[pl.ds(0, CH)], dst2d[b].at[k],
                                  dsem[b]).wait()

    def fire_gathers(b):
        for k in range(K):
            pltpu.async_copy(h_hbm.at[srcidx[b].at[pl.ds(k * CH, CH)]],
                             rows[b].at[pl.ds(k * CH, CH), :], gsem[b])

    def drain_gathers(b):
        for k in range(K):
            pltpu.make_async_copy(h_hbm.at[srcidx[b].at[pl.ds(k * CH, CH)]],
                                  rows[b].at[pl.ds(k * CH, CH), :],
                                  gsem[b]).wait()

    def fire_scatters(b):
        for k in range(K):
            pltpu.async_copy(rows[b].at[pl.ds(k * CH, CH), :],
                             agg.at[dst2d[b].at[k]], ssem[b], add=True)

    def drain_scatters(b):
        for k in range(K):
            pltpu.make_async_copy(rows[b].at[pl.ds(k * CH, CH), :],
                                  agg.at[dst2d[b].at[k]], ssem[b]).wait()

    def compute(b):
        attr_b = attr[b]
        rows_b = rows[b]
        wcs = tuple(tuple(wct_v[k, pl.ds(v * 16, 16)] for v in range(8))
                    for k in range(4))

        def ebody(c, wcs):
            av = attr_b[pl.ds(c * 4, 16)]
            a0 = av[0]
            a1 = av[1]
            a2 = av[2]
            a3 = av[3]
            for v in range(8):
                sl = pl.ds(v * 16, 16)
                ea = (a0 * wcs[0][v] + a1 * wcs[1][v]
                      + a2 * wcs[2][v] + a3 * wcs[3][v])
                rows_b[c, sl] = rows_b[c, sl] * ea
            return wcs
        lax.fori_loop(0, KCH, ebody, wcs)

    pltpu.sync_copy(src_hbm.at[pl.ds(ebase, KCH)], srcidx[0])
    pltpu.sync_copy(ea_hbm.at[pl.ds(ebase * 4, KCH * 4)],
                    attr[0].at[pl.ds(0, KCH * 4)])
    fire_gathers(0)
    fire_dst(0, 0)
    fire_src_attr(1, 1)

    def pair_body(m, carry):
        @pl.when(m > 0)
        def _():
            drain_scatters(1)
        fire_dst(2 * m + 1, 1)
        wait_src_attr(1)
        fire_gathers(1)
        drain_gathers(0)
        compute(0)

        @pl.when(m < NSUPER // 2 - 1)
        def _():
            fire_src_attr(2 * m + 2, 0)
        wait_dst(0)
        fire_scatters(0)

        drain_scatters(0)

        @pl.when(m < NSUPER // 2 - 1)
        def _():
            fire_dst(2 * m + 2, 0)
            wait_src_attr(0)
            fire_gathers(0)
        drain_gathers(1)
        compute(1)

        @pl.when(m < NSUPER // 2 - 1)
        def _():
            fire_src_attr(2 * m + 3, 1)
        wait_dst(1)
        fire_scatters(1)
        return carry

    lax.fori_loop(0, NSUPER // 2, pair_body, 0)
    drain_scatters(1)
    plsc.subcore_barrier()
    pltpu.sync_copy(agg.at[pl.ds(sid * RPT, RPT), :],
                    out_hbm.at[cid, pl.ds(sid * RPT, RPT), :])


def _embed(tab_hbm, idx_hbm, out_hbm, idxv, rowsv, sem):
    cid = lax.axis_index("c")
    sid = lax.axis_index("s")
    wid = sid * NC + cid
    for j in range(RPW // CH):
        base = wid * RPW + j * CH
        pltpu.sync_copy(idx_hbm.at[pl.ds(base, CH)], idxv)
        pltpu.async_copy(tab_hbm.at[idxv], rowsv, sem).wait()
        pltpu.sync_copy(rowsv, out_hbm.at[pl.ds(base, CH), :])


def _fold(wemb_ref, wel_ref, out_ref):
    for l in range(L):
        out_ref[l] = lax.dot_general(
            wemb_ref[...], wel_ref[l], (((0,), (1,)), ((), ())),
            preferred_element_type=jnp.float32)


def _layer(p_ref, h_ref, wrel_ref, brel_ref, wroot_ref, wres_ref, out_ref):
    agg = p_ref[0] + p_ref[1]
    h = h_ref[...]
    t = lax.dot_general(agg, wrel_ref[...], (((1,), (1,)), ((), ())),
                        preferred_element_type=jnp.float32) + brel_ref[...]
    t = t + lax.dot_general(h, wroot_ref[...], (((1,), (1,)), ((), ())),
                            preferred_element_type=jnp.float32)
    t = jnp.maximum(t, 0.0)
    out_ref[...] = t + lax.dot_general(h, wres_ref[...], (((1,), (1,)), ((), ())),
                                       preferred_element_type=jnp.float32)


def _head(h_ref, w1_ref, b1_ref, w2_ref, b2_ref, out_ref):
    t = lax.dot_general(h_ref[...], w1_ref[...], (((1,), (1,)), ((), ())),
                        preferred_element_type=jnp.float32) + b1_ref[...]
    g = 0.5 * t * (1.0 + lax.erf(t * (1.0 / math.sqrt(2.0))))
    out_ref[...] = lax.dot_general(g, w2_ref[...], (((1,), (1,)), ((), ())),
                                   preferred_element_type=jnp.float32) + b2_ref[...]


BM = 1000


def kernel(x, edge_index, edge_attr, vert_emb, W_edge_emb, W_el, W_rel, b_rel,
           W_root, W_res, W_h1, b_h1, W_h2, b_h2):
    f32 = jnp.float32
    x_pad = jnp.pad(x.astype(jnp.int32), (0, NPAD - N))
    src = jnp.pad(edge_index[0].astype(jnp.int32), (0, E_PAD - E))
    dst = jnp.pad(edge_index[1].astype(jnp.int32), (0, E_PAD - E))
    ea_flat = jnp.pad(edge_attr.reshape(E * 4), (0, (E_PAD - E) * 4))

    wct_all = pl.pallas_call(
        _fold,
        out_shape=jax.ShapeDtypeStruct((L, 4, H), f32),
    )(W_edge_emb, W_el)

    mesh = plsc.VectorSubcoreMesh(core_axis_name="c", subcore_axis_name="s",
                                  num_cores=NC, num_subcores=NS)

    embed_call = pl.kernel(
        _embed,
        out_type=jax.ShapeDtypeStruct((NPAD, H), f32),
        mesh=mesh,
        scratch_types=[
            pltpu.VMEM((CH,), jnp.int32),
            pltpu.VMEM((CH, H), f32),
            pltpu.SemaphoreType.DMA,
        ],
    )
    h = embed_call(vert_emb, x_pad)[:N]

    edge_call = pl.kernel(
        _edge_pass,
        out_type=jax.ShapeDtypeStruct((NC, NPAD, H), f32),
        mesh=mesh,
        scratch_types=[
            pltpu.VMEM((KCH,), jnp.int32),
            pltpu.VMEM((KCH,), jnp.int32),
            pltpu.VMEM((K, CH), jnp.int32),
            pltpu.VMEM((K, CH), jnp.int32),
            pltpu.VMEM((KCH * 4 + 16,), f32),
            pltpu.VMEM((KCH * 4 + 16,), f32),
            pltpu.VMEM((KCH, H), f32),
            pltpu.VMEM((KCH, H), f32),
            pltpu.VMEM((4, H), f32),
            pltpu.VMEM((ZR, H), f32),
            pltpu.VMEM_SHARED((NPAD, H), f32),
        ] + [pltpu.SemaphoreType.DMA] * 8,
    )

    layer_call = pl.pallas_call(
        _layer,
        grid=(N // BM,),
        in_specs=[
            pl.BlockSpec((NC, BM, H), lambda m: (0, m, 0)),
            pl.BlockSpec((BM, H), lambda m: (m, 0)),
            pl.BlockSpec((H, H), lambda m: (0, 0)),
            pl.BlockSpec((1, H), lambda m: (0, 0)),
            pl.BlockSpec((H, H), lambda m: (0, 0)),
            pl.BlockSpec((H, H), lambda m: (0, 0)),
        ],
        out_specs=pl.BlockSpec((BM, H), lambda m: (m, 0)),
        out_shape=jax.ShapeDtypeStruct((N, H), f32),
    )

    for l in range(L):
        p = edge_call(h, src, dst, ea_flat, wct_all[l])
        h = layer_call(p, h, W_rel[l], b_rel[l].reshape(1, H),
                       W_root[l], W_res[l])

    w2_pad = jnp.pad(W_h2, ((0, H - 1), (0, 0)))
    b2_pad = jnp.pad(b2_col := b_h2.reshape(1, 1), ((0, 0), (0, H - 1)))
    y = pl.pallas_call(
        _head,
        grid=(N // BM,),
        in_specs=[
            pl.BlockSpec((BM, H), lambda m: (m, 0)),
            pl.BlockSpec((NFF, H), lambda m: (0, 0)),
            pl.BlockSpec((1, NFF), lambda m: (0, 0)),
            pl.BlockSpec((H, NFF), lambda m: (0, 0)),
            pl.BlockSpec((1, H), lambda m: (0, 0)),
        ],
        out_specs=pl.BlockSpec((BM, H), lambda m: (m, 0)),
        out_shape=jax.ShapeDtypeStruct((N, H), f32),
    )(h, W_h1, b_h1.reshape(1, NFF), w2_pad, b2_pad)
    return y[:, :1]
